# Initial kernel scaffold; baseline (speedup 1.0000x reference)
import jax
import jax.numpy as jnp
from jax.experimental import pallas as pl


def kernel(ids, table, W, b):
    raise NotImplementedError("write your pallas kernel here")


# speedup vs baseline: 1.6288x; 1.6288x over previous
import functools

import jax
import jax.numpy as jnp
from jax import lax
from jax.experimental import pallas as pl
from jax.experimental.pallas import tpu as pltpu
from jax.experimental.pallas import tpu_sc as plsc

NC = 2
NS = 16
LANES = 16


def _sc_sum_pool(table, ids2, B, HALF, D):
    NW = NC * NS
    RPW = B // NW
    mesh = plsc.VectorSubcoreMesh(core_axis_name="c", subcore_axis_name="s")
    CG = D // LANES

    @functools.partial(
        pl.kernel,
        out_type=jax.ShapeDtypeStruct((B, D), jnp.float32),
        mesh=mesh,
        scratch_types=[
            pltpu.VMEM((2 * RPW, HALF), jnp.int32),
            pltpu.VMEM((HALF, D), jnp.float32),
            pltpu.VMEM((HALF, D), jnp.float32),
            pltpu.VMEM((RPW, D), jnp.float32),
            pltpu.SemaphoreType.DMA,
            pltpu.SemaphoreType.DMA,
        ],
    )
    def sc_kernel(table_hbm, ids_hbm, out_hbm, ids_v, buf0, buf1, acc_v, sem0, sem1):
        wid = lax.axis_index("s") * NC + lax.axis_index("c")
        base = wid * (2 * RPW)
        pltpu.sync_copy(ids_hbm.at[pl.ds(base, 2 * RPW)], ids_v)

        @pl.loop(0, RPW)
        def _(r):
            h0 = pltpu.async_copy(table_hbm.at[ids_v.at[2 * r]], buf0, sem0)
            h1 = pltpu.async_copy(table_hbm.at[ids_v.at[2 * r + 1]], buf1, sem1)
            h0.wait()
            h1.wait()

            def body(j, accs):
                return tuple(
                    accs[c]
                    + buf0[j, pl.ds(c * LANES, LANES)]
                    + buf1[j, pl.ds(c * LANES, LANES)]
                    for c in range(CG)
                )

            zero = jnp.zeros((LANES,), jnp.float32)
            accs = lax.fori_loop(0, HALF, body, (zero,) * CG)
            for c in range(CG):
                acc_v[r, pl.ds(c * LANES, LANES)] = accs[c]

        pltpu.sync_copy(acc_v, out_hbm.at[pl.ds(wid * RPW, RPW)])

    return sc_kernel(table, ids2)


def _tc_finish(ids, sums, Wt, b2, B, SEQ, D):
    def tc_body(ids_ref, sums_ref, wt_ref, b_ref, out_ref):
        idv = ids_ref[...]
        cnt = jnp.sum((idv != 0).astype(jnp.float32), axis=1, keepdims=True)
        pooled = sums_ref[...] / jnp.maximum(cnt, 1.0)
        h = jnp.maximum(pooled, 0.0)
        h = lax.dot_general(h, wt_ref[...], (((1,), (0,)), ((), ())),
                            preferred_element_type=jnp.float32)
        h = h + b_ref[...]
        nrm = jnp.maximum(jnp.sqrt(jnp.sum(h * h, axis=1, keepdims=True)), 1e-12)
        out_ref[...] = h / nrm

    return pl.pallas_call(
        tc_body,
        out_shape=jax.ShapeDtypeStruct((B, D), jnp.float32),
    )(ids, sums, Wt, b2)


def kernel(ids, table, W, b):
    B, SEQ = ids.shape
    V, D = table.shape
    HALF = SEQ // 2
    ids2 = ids.reshape(B * 2, HALF)
    sums = _sc_sum_pool(table, ids2, B, HALF, D)
    return _tc_finish(ids, sums, W.T, b.reshape(1, D), B, SEQ, D)


# speedup vs baseline: 2.2142x; 1.3594x over previous
import functools

import jax
import jax.numpy as jnp
from jax import lax
from jax.experimental import pallas as pl
from jax.experimental.pallas import tpu as pltpu
from jax.experimental.pallas import tpu_sc as plsc

NC = 2
NS = 16
LANES = 16


def _sc_sum_pool(table, ids2, B, HALF, D):
    NW = NC * NS
    RPW = B // NW
    mesh = plsc.VectorSubcoreMesh(core_axis_name="c", subcore_axis_name="s")
    CG = D // LANES

    @functools.partial(
        pl.kernel,
        out_type=jax.ShapeDtypeStruct((B, D), jnp.float32),
        mesh=mesh,
        scratch_types=[
            pltpu.VMEM((2 * RPW, HALF), jnp.int32),
            pltpu.VMEM((HALF, D), jnp.float32),
            pltpu.VMEM((HALF, D), jnp.float32),
            pltpu.VMEM((HALF, D), jnp.float32),
            pltpu.VMEM((HALF, D), jnp.float32),
            pltpu.VMEM((RPW, D), jnp.float32),
            pltpu.SemaphoreType.DMA,
            pltpu.SemaphoreType.DMA,
        ],
    )
    def sc_kernel(table_hbm, ids_hbm, out_hbm, ids_v,
                  bufa0, bufa1, bufb0, bufb1, acc_v, sema, semb):
        wid = lax.axis_index("s") * NC + lax.axis_index("c")
        base = wid * (2 * RPW)
        pltpu.sync_copy(ids_hbm.at[pl.ds(base, 2 * RPW)], ids_v)

        def issue(b0, b1, sem, r):
            pltpu.async_copy(table_hbm.at[ids_v.at[2 * r]], b0, sem)
            pltpu.async_copy(table_hbm.at[ids_v.at[2 * r + 1]], b1, sem)

        def wait(b0, b1, sem, r):
            pltpu.make_async_copy(table_hbm.at[ids_v.at[2 * r]], b0, sem).wait()
            pltpu.make_async_copy(table_hbm.at[ids_v.at[2 * r + 1]], b1, sem).wait()

        def accumulate(b0, b1, r):
            def body(j, accs):
                return tuple(
                    accs[c]
                    + b0[j, pl.ds(c * LANES, LANES)]
                    + b1[j, pl.ds(c * LANES, LANES)]
                    for c in range(CG)
                )

            zero = jnp.zeros((LANES,), jnp.float32)
            accs = lax.fori_loop(0, HALF, body, (zero,) * CG)
            for c in range(CG):
                acc_v[r, pl.ds(c * LANES, LANES)] = accs[c]

        issue(bufa0, bufa1, sema, 0)

        @pl.loop(0, RPW, step=2)
        def _(r):
            wait(bufa0, bufa1, sema, r)
            issue(bufb0, bufb1, semb, r + 1)
            accumulate(bufa0, bufa1, r)
            wait(bufb0, bufb1, semb, r + 1)

            @pl.when(r + 2 < RPW)
            def _():
                issue(bufa0, bufa1, sema, r + 2)

            accumulate(bufb0, bufb1, r + 1)

        pltpu.sync_copy(acc_v, out_hbm.at[pl.ds(wid * RPW, RPW)])

    return sc_kernel(table, ids2)


def _tc_finish(ids, sums, Wt, b2, B, SEQ, D):
    def tc_body(ids_ref, sums_ref, wt_ref, b_ref, out_ref):
        idv = ids_ref[...]
        cnt = jnp.sum((idv != 0).astype(jnp.float32), axis=1, keepdims=True)
        pooled = sums_ref[...] / jnp.maximum(cnt, 1.0)
        h = jnp.maximum(pooled, 0.0)
        h = lax.dot_general(h, wt_ref[...], (((1,), (0,)), ((), ())),
                            preferred_element_type=jnp.float32)
        h = h + b_ref[...]
        nrm = jnp.maximum(jnp.sqrt(jnp.sum(h * h, axis=1, keepdims=True)), 1e-12)
        out_ref[...] = h / nrm

    return pl.pallas_call(
        tc_body,
        out_shape=jax.ShapeDtypeStruct((B, D), jnp.float32),
    )(ids, sums, Wt, b2)


def kernel(ids, table, W, b):
    B, SEQ = ids.shape
    V, D = table.shape
    HALF = SEQ // 2
    ids2 = ids.reshape(B * 2, HALF)
    sums = _sc_sum_pool(table, ids2, B, HALF, D)
    return _tc_finish(ids, sums, W.T, b.reshape(1, D), B, SEQ, D)


# speedup vs baseline: 3.2654x; 1.4748x over previous
import functools

import jax
import jax.numpy as jnp
from jax import lax
from jax.experimental import pallas as pl
from jax.experimental.pallas import tpu as pltpu
from jax.experimental.pallas import tpu_sc as plsc

NC = 2
NS = 16
LANES = 16


def _sc_sum_pool(table, ids2, B, HALF, D):
    NW = NC * NS
    RPW = B // NW
    mesh = plsc.VectorSubcoreMesh(core_axis_name="c", subcore_axis_name="s")
    CG = D // LANES

    NBUF = 4
    NWIN = 2 * RPW

    @functools.partial(
        pl.kernel,
        out_type=jax.ShapeDtypeStruct((B, D), jnp.float32),
        mesh=mesh,
        scratch_types=[
            pltpu.VMEM((2 * RPW, HALF), jnp.int32),
        ] + [pltpu.VMEM((HALF, D), jnp.float32) for _ in range(NBUF)] + [
            pltpu.VMEM((RPW, D), jnp.float32),
        ] + [pltpu.SemaphoreType.DMA for _ in range(NBUF)],
    )
    def sc_kernel(table_hbm, ids_hbm, out_hbm, ids_v, *rest):
        bufs = rest[:NBUF]
        acc_v = rest[NBUF]
        sems = rest[NBUF + 1:]
        wid = lax.axis_index("s") * NC + lax.axis_index("c")
        base = wid * NWIN
        pltpu.sync_copy(ids_hbm.at[pl.ds(base, NWIN)], ids_v)

        def issue(b, w):
            pltpu.async_copy(table_hbm.at[ids_v.at[w]], bufs[b], sems[b])

        def wait(b, w):
            pltpu.make_async_copy(table_hbm.at[ids_v.at[w]], bufs[b], sems[b]).wait()

        def accumulate(buf, accs):
            def body(j, a):
                return tuple(
                    a[c] + buf[j, pl.ds(c * LANES, LANES)] for c in range(CG)
                )

            return lax.fori_loop(0, HALF, body, accs)

        for b in range(NBUF - 1):
            issue(b, b)

        zeros = (jnp.zeros((LANES,), jnp.float32),) * CG

        @pl.loop(0, NWIN, step=NBUF)
        def _(w0):
            accs = zeros
            for b in range(NBUF):
                w = w0 + b
                wait(b, w)

                @pl.when(w + NBUF - 1 < NWIN)
                def _():
                    issue((b + NBUF - 1) % NBUF, w + NBUF - 1)

                accs = accumulate(bufs[b], accs)
                if b % 2 == 1:
                    r = (w0 + b - 1) // 2
                    for c in range(CG):
                        acc_v[r, pl.ds(c * LANES, LANES)] = accs[c]
                    accs = zeros

        pltpu.sync_copy(acc_v, out_hbm.at[pl.ds(wid * RPW, RPW)])

    return sc_kernel(table, ids2)


def _tc_finish(ids, sums, Wt, b2, B, SEQ, D):
    def tc_body(ids_ref, sums_ref, wt_ref, b_ref, out_ref):
        idv = ids_ref[...]
        cnt = jnp.sum((idv != 0).astype(jnp.float32), axis=1, keepdims=True)
        pooled = sums_ref[...] / jnp.maximum(cnt, 1.0)
        h = jnp.maximum(pooled, 0.0)
        h = lax.dot_general(h, wt_ref[...], (((1,), (0,)), ((), ())),
                            preferred_element_type=jnp.float32)
        h = h + b_ref[...]
        nrm = jnp.maximum(jnp.sqrt(jnp.sum(h * h, axis=1, keepdims=True)), 1e-12)
        out_ref[...] = h / nrm

    return pl.pallas_call(
        tc_body,
        out_shape=jax.ShapeDtypeStruct((B, D), jnp.float32),
    )(ids, sums, Wt, b2)


def kernel(ids, table, W, b):
    B, SEQ = ids.shape
    V, D = table.shape
    HALF = SEQ // 2
    ids2 = ids.reshape(B * 2, HALF)
    sums = _sc_sum_pool(table, ids2, B, HALF, D)
    return _tc_finish(ids, sums, W.T, b.reshape(1, D), B, SEQ, D)


# speedup vs baseline: 3.3270x; 1.0189x over previous
import functools

import jax
import jax.numpy as jnp
from jax import lax
from jax.experimental import pallas as pl
from jax.experimental.pallas import tpu as pltpu
from jax.experimental.pallas import tpu_sc as plsc

NC = 2
NS = 16
LANES = 16
WIN = 50
NBUF = 8


def _sc_sum_pool(table, ids2, B, SEQ, D):
    NW = NC * NS
    RPW = B // NW
    WPR = SEQ // WIN
    NWIN = RPW * WPR
    assert NWIN % NBUF == 0 and NBUF % WPR == 0
    mesh = plsc.VectorSubcoreMesh(core_axis_name="c", subcore_axis_name="s")
    CG = D // LANES

    @functools.partial(
        pl.kernel,
        out_type=jax.ShapeDtypeStruct((B, D), jnp.float32),
        mesh=mesh,
        scratch_types=[
            pltpu.VMEM((NWIN, WIN), jnp.int32),
        ] + [pltpu.VMEM((WIN, D), jnp.float32) for _ in range(NBUF)] + [
            pltpu.VMEM((2, D), jnp.float32),
        ] + [pltpu.SemaphoreType.DMA for _ in range(NBUF + 1)],
    )
    def sc_kernel(table_hbm, ids_hbm, out_hbm, ids_v, *rest):
        bufs = rest[:NBUF]
        out_stage = rest[NBUF]
        sems = rest[NBUF + 1:NBUF + 1 + NBUF]
        osem = rest[NBUF + 1 + NBUF]
        wid = lax.axis_index("s") * NC + lax.axis_index("c")
        base = wid * NWIN
        pltpu.sync_copy(ids_hbm.at[pl.ds(base, NWIN)], ids_v)

        def issue(b, w):
            pltpu.async_copy(table_hbm.at[ids_v.at[w]], bufs[b], sems[b])

        def wait(b, w):
            pltpu.make_async_copy(table_hbm.at[ids_v.at[w]], bufs[b], sems[b]).wait()

        def accumulate(buf, accs):
            def body(j, a):
                return tuple(
                    a[c] + buf[j, pl.ds(c * LANES, LANES)] for c in range(CG)
                )

            return lax.fori_loop(0, WIN, body, accs)

        for b in range(NBUF - 1):
            issue(b, b)

        zeros = (jnp.zeros((LANES,), jnp.float32),) * CG

        @pl.loop(0, NWIN, step=NBUF)
        def _(w0):
            accs = zeros
            for b in range(NBUF):
                w = w0 + b
                wait(b, w)

                @pl.when(w + NBUF - 1 < NWIN)
                def _():
                    issue((b + NBUF - 1) % NBUF, w + NBUF - 1)

                accs = accumulate(bufs[b], accs)
                if b % WPR == WPR - 1:
                    r = (w0 + b) // WPR
                    p = ((b + 1) // WPR - 1) % 2

                    @pl.when(r >= 2)
                    def _():
                        pltpu.make_async_copy(
                            out_stage.at[pl.ds(p, 1)],
                            out_hbm.at[pl.ds(wid * RPW + r - 2, 1)],
                            osem,
                        ).wait()

                    for c in range(CG):
                        out_stage[p, pl.ds(c * LANES, LANES)] = accs[c]
                    pltpu.async_copy(
                        out_stage.at[pl.ds(p, 1)],
                        out_hbm.at[pl.ds(wid * RPW + r, 1)],
                        osem,
                    )
                    accs = zeros

        for p, r in ((0, RPW - 2), (1, RPW - 1)):
            pltpu.make_async_copy(
                out_stage.at[pl.ds(p, 1)],
                out_hbm.at[pl.ds(wid * RPW + r, 1)],
                osem,
            ).wait()

    return sc_kernel(table, ids2)


def _tc_finish(ids, sums, Wt, b2, B, SEQ, D):
    def tc_body(ids_ref, sums_ref, wt_ref, b_ref, out_ref):
        idv = ids_ref[...]
        cnt = jnp.sum((idv != 0).astype(jnp.float32), axis=1, keepdims=True)
        pooled = sums_ref[...] / jnp.maximum(cnt, 1.0)
        h = jnp.maximum(pooled, 0.0)
        h = lax.dot_general(h, wt_ref[...], (((1,), (0,)), ((), ())),
                            preferred_element_type=jnp.float32)
        h = h + b_ref[...]
        nrm = jnp.maximum(jnp.sqrt(jnp.sum(h * h, axis=1, keepdims=True)), 1e-12)
        out_ref[...] = h / nrm

    return pl.pallas_call(
        tc_body,
        out_shape=jax.ShapeDtypeStruct((B, D), jnp.float32),
    )(ids, sums, Wt, b2)


def kernel(ids, table, W, b):
    B, SEQ = ids.shape
    V, D = table.shape
    ids2 = ids.reshape(B * (SEQ // WIN), WIN)
    sums = _sc_sum_pool(table, ids2, B, SEQ, D)
    return _tc_finish(ids, sums, W.T, b.reshape(1, D), B, SEQ, D)
